# baseline (speedup 1.0000x reference)
import functools

import jax
import jax.numpy as jnp
from jax import lax
from jax.experimental import pallas as pl
from jax.experimental.pallas import tpu as pltpu
from jax.experimental.pallas import tpu_sc as plsc

NUM_EMBEDDINGS = 1000000
D = 64
BATCH = 16384
HIST = 20
B = BATCH * HIST

NC = 2
NS = 16
NW = NC * NS

CW = 128
TBW = 4
NCHUNK = HIST * TBW
NBUF = 4

_mesh = plsc.VectorSubcoreMesh(
    core_axis_name="c", subcore_axis_name="s", num_cores=NC, num_subcores=NS
)


@functools.partial(
    pl.kernel,
    out_type=jax.ShapeDtypeStruct((HIST, 8, 128, 8, 128), jnp.float32),
    mesh=_mesh,
    compiler_params=pltpu.CompilerParams(
        use_tc_tiling_on_sc=True, needs_layout_passes=False
    ),
    scratch_types=[
        pltpu.VMEM((HIST, 4 * CW), jnp.int32),
        pltpu.VMEM((NBUF, CW), jnp.int32),
        pltpu.VMEM((NBUF, CW, 128), jnp.float32),
        pltpu.VMEM((NBUF, 8, 8, CW), jnp.float32),
        pltpu.SemaphoreType.DMA((NBUF,)),
        pltpu.SemaphoreType.DMA((NBUF,)),
    ],
)
def _gather_kernel(ids_hbm, table_hbm, out_hbm, ibuf, gbuf, rbuf, obuf,
                   gsem, osem):
    wid = lax.axis_index("s") * NC + lax.axis_index("c")
    b0 = wid * (4 * CW)

    iota = jax.lax.iota(jnp.int32, 16)

    for h in range(HIST):
        pltpu.sync_copy(
            ids_hbm.at[pl.ds(h * BATCH + b0, 4 * CW)], ibuf.at[h]
        )

    def shift_ids(s, q):
        h = q // TBW
        j = lax.rem(q, TBW)
        for k in range(CW // 16):
            gbuf[s, pl.ds(16 * k, 16)] = (
                ibuf[h, pl.ds(128 * j + 16 * k, 16)] >> 1
            )

    def gather(s):
        return pltpu.make_async_copy(
            table_hbm.at[gbuf.at[s]], rbuf.at[s], gsem.at[s]
        )

    def out_copy(s, q):
        h = q // TBW
        tb = TBW * wid + lax.rem(q, TBW)
        return pltpu.make_async_copy(
            obuf.at[s], out_hbm.at[h, :, tb], osem.at[s]
        )

    def extract(s, q):
        h = q // TBW
        j = lax.rem(q, TBW)
        for m in range(CW // 16):
            tv = ibuf[h, pl.ds(128 * j + 16 * m, 16)]
            parv = (tv & 1) * 64
            rowv = iota + 16 * m

            @plsc.parallel_loop(0, D, unroll=2)
            def _feat(c):
                colv = parv + c
                v = plsc.load_gather(rbuf.at[s], [rowv, colv])
                obuf[s, c // 8, lax.rem(c, 8), pl.ds(16 * m, 16)] = v

    for s in range(NBUF):
        shift_ids(s, s)
        gather(s).start()

    @pl.loop(0, NCHUNK, step=NBUF)
    def _round(q0):
        for s in range(NBUF):
            q = q0 + s
            gather(s).wait()

            @pl.when(q >= NBUF)
            def _wait_out():
                out_copy(s, q - NBUF).wait()

            extract(s, q)
            out_copy(s, q).start()
            nq = q + NBUF

            @pl.when(nq < NCHUNK)
            def _refill():
                shift_ids(s, nq)
                gather(s).start()

    for s in range(NBUF):
        out_copy(s, NCHUNK - NBUF + s).wait()


def kernel(token_ids, weight):
    ids_flat = token_ids.T.reshape(B)
    t128 = weight.reshape(NUM_EMBEDDINGS // 2, 128)
    out5d = _gather_kernel(ids_flat, t128)
    out = out5d.transpose(2, 4, 0, 1, 3).reshape(BATCH, HIST, D)
    return out


# speedup vs baseline: 1.2532x; 1.2532x over previous
import functools

import jax
import jax.numpy as jnp
from jax import lax
from jax.experimental import pallas as pl
from jax.experimental.pallas import tpu as pltpu
from jax.experimental.pallas import tpu_sc as plsc

NUM_EMBEDDINGS = 1000000
D = 64
BATCH = 16384
HIST = 20
B = BATCH * HIST

NC = 2
NS = 16
NW = NC * NS

CW = 128
TBW = 4
NCHUNK = HIST * TBW
NBUF = 4

_mesh = plsc.VectorSubcoreMesh(
    core_axis_name="c", subcore_axis_name="s", num_cores=NC, num_subcores=NS
)


@functools.partial(
    pl.kernel,
    out_type=jax.ShapeDtypeStruct((HIST, 8, 128, 8, 128), jnp.float32),
    mesh=_mesh,
    compiler_params=pltpu.CompilerParams(
        use_tc_tiling_on_sc=True, needs_layout_passes=False
    ),
    scratch_types=[
        pltpu.VMEM((HIST, 4 * CW), jnp.int32),
        pltpu.VMEM((NBUF, CW), jnp.int32),
        pltpu.VMEM((NBUF, CW, 128), jnp.float32),
        pltpu.VMEM((NBUF, 8, 8, CW), jnp.float32),
        pltpu.SemaphoreType.DMA((NBUF,)),
        pltpu.SemaphoreType.DMA((NBUF,)),
    ],
)
def _gather_kernel(ids_hbm, table_hbm, out_hbm, ibuf, gbuf, rbuf, obuf,
                   gsem, osem):
    wid = lax.axis_index("s") * NC + lax.axis_index("c")
    b0 = wid * (4 * CW)

    iota = jax.lax.iota(jnp.int32, 16)

    for h in range(HIST):
        pltpu.sync_copy(
            ids_hbm.at[pl.ds(h * BATCH + b0, 4 * CW)], ibuf.at[h]
        )

    def shift_ids(s, q):
        h = q // TBW
        j = lax.rem(q, TBW)
        for k in range(CW // 16):
            gbuf[s, pl.ds(16 * k, 16)] = (
                ibuf[h, pl.ds(128 * j + 16 * k, 16)] >> 1
            )

    def gather(s):
        return pltpu.make_async_copy(
            table_hbm.at[gbuf.at[s]], rbuf.at[s], gsem.at[s]
        )

    def out_copy(s, q):
        h = q // TBW
        tb = TBW * wid + lax.rem(q, TBW)
        return pltpu.make_async_copy(
            obuf.at[s], out_hbm.at[h, :, tb], osem.at[s]
        )

    def extract(s, q):
        h = q // TBW
        j = lax.rem(q, TBW)
        for m in range(CW // 16):
            tv = ibuf[h, pl.ds(128 * j + 16 * m, 16)]
            parv = (tv & 1) * 64
            rowv = iota + 16 * m

            @plsc.parallel_loop(0, D, unroll=2)
            def _feat(c0):
                cvec = (c0 + iota) & 63
                v = plsc.load_gather(rbuf.at[s], [rowv, parv + cvec])
                plsc.store_scatter(
                    obuf.at[s], [cvec >> 3, cvec & 7, rowv], v
                )

    for s in range(NBUF):
        shift_ids(s, s)
        gather(s).start()

    @pl.loop(0, NCHUNK, step=NBUF)
    def _round(q0):
        for s in range(NBUF):
            q = q0 + s
            gather(s).wait()

            @pl.when(q >= NBUF)
            def _wait_out():
                out_copy(s, q - NBUF).wait()

            extract(s, q)
            out_copy(s, q).start()
            nq = q + NBUF

            @pl.when(nq < NCHUNK)
            def _refill():
                shift_ids(s, nq)
                gather(s).start()

    for s in range(NBUF):
        out_copy(s, NCHUNK - NBUF + s).wait()


def kernel(token_ids, weight):
    ids_flat = token_ids.T.reshape(B)
    t128 = weight.reshape(NUM_EMBEDDINGS // 2, 128)
    out5d = _gather_kernel(ids_flat, t128)
    out = out5d.transpose(2, 4, 0, 1, 3).reshape(BATCH, HIST, D)
    return out


# speedup vs baseline: 1.3773x; 1.0990x over previous
import functools

import jax
import jax.numpy as jnp
from jax import lax
from jax.experimental import pallas as pl
from jax.experimental.pallas import tpu as pltpu
from jax.experimental.pallas import tpu_sc as plsc

NUM_EMBEDDINGS = 1000000
D = 64
BATCH = 16384
HIST = 20
B = BATCH * HIST

NC = 2
NS = 16
NW = NC * NS

CW = 128
TBW = 4
NCHUNK = HIST * TBW
NBUF = 4

_mesh = plsc.VectorSubcoreMesh(
    core_axis_name="c", subcore_axis_name="s", num_cores=NC, num_subcores=NS
)


@functools.partial(
    pl.kernel,
    out_type=jax.ShapeDtypeStruct((HIST, 8, 128, 8, 128), jnp.float32),
    mesh=_mesh,
    compiler_params=pltpu.CompilerParams(
        use_tc_tiling_on_sc=True, needs_layout_passes=False
    ),
    scratch_types=[
        pltpu.VMEM((HIST, 4 * CW), jnp.int32),
        pltpu.VMEM((NBUF, CW), jnp.int32),
        pltpu.VMEM((NBUF, CW, 128), jnp.float32),
        pltpu.VMEM((NBUF, 8, 8, CW), jnp.float32),
        pltpu.SemaphoreType.DMA((NBUF,)),
        pltpu.SemaphoreType.DMA((NBUF,)),
    ],
)
def _gather_kernel(ids_hbm, table_hbm, out_hbm, ibuf, gbuf, rbuf, obuf,
                   gsem, osem):
    wid = lax.axis_index("s") * NC + lax.axis_index("c")
    b0 = wid * (4 * CW)

    iota = jax.lax.iota(jnp.int32, 16)

    for h in range(HIST):
        pltpu.sync_copy(
            ids_hbm.at[pl.ds(h * BATCH + b0, 4 * CW)], ibuf.at[h]
        )

    def shift_ids(s, q):
        h = q // TBW
        j = lax.rem(q, TBW)
        for k in range(CW // 16):
            gbuf[s, pl.ds(16 * k, 16)] = ibuf[h, pl.ds(128 * j + 16 * k, 16)]

    def gather(s):
        return pltpu.make_async_copy(
            table_hbm.at[gbuf.at[s]], rbuf.at[s], gsem.at[s]
        )

    def out_copy(s, q):
        h = q // TBW
        tb = TBW * wid + lax.rem(q, TBW)
        return pltpu.make_async_copy(
            obuf.at[s], out_hbm.at[h, :, tb], osem.at[s]
        )

    def extract(s, q):
        h = q // TBW
        j = lax.rem(q, TBW)
        for m in range(CW // 16):
            rowv = iota + 16 * m

            @plsc.parallel_loop(0, D, unroll=2)
            def _feat(c0):
                cvec = (c0 + iota) & 63
                v = plsc.load_gather(rbuf.at[s], [rowv, cvec])
                plsc.store_scatter(
                    obuf.at[s], [cvec >> 3, cvec & 7, rowv], v
                )

    for s in range(NBUF):
        shift_ids(s, s)
        gather(s).start()

    @pl.loop(0, NCHUNK, step=NBUF)
    def _round(q0):
        for s in range(NBUF):
            q = q0 + s
            gather(s).wait()

            @pl.when(q >= NBUF)
            def _wait_out():
                out_copy(s, q - NBUF).wait()

            extract(s, q)
            out_copy(s, q).start()
            nq = q + NBUF

            @pl.when(nq < NCHUNK)
            def _refill():
                shift_ids(s, nq)
                gather(s).start()

    for s in range(NBUF):
        out_copy(s, NCHUNK - NBUF + s).wait()


def kernel(token_ids, weight):
    ids_flat = token_ids.T.reshape(B)
    t128 = jnp.pad(weight, ((0, 0), (0, D)))
    out5d = _gather_kernel(ids_flat, t128)
    out = out5d.transpose(2, 4, 0, 1, 3).reshape(BATCH, HIST, D)
    return out


# speedup vs baseline: 1.3789x; 1.0012x over previous
import functools

import jax
import jax.numpy as jnp
from jax import lax
from jax.experimental import pallas as pl
from jax.experimental.pallas import tpu as pltpu
from jax.experimental.pallas import tpu_sc as plsc

NUM_EMBEDDINGS = 1000000
D = 64
BATCH = 16384
HIST = 20
B = BATCH * HIST

NC = 2
NS = 16
NW = NC * NS

CW = 128
TBW = 4
NCHUNK = HIST * TBW
NBUF = 4

_mesh = plsc.VectorSubcoreMesh(
    core_axis_name="c", subcore_axis_name="s", num_cores=NC, num_subcores=NS
)


@functools.partial(
    pl.kernel,
    out_type=jax.ShapeDtypeStruct((HIST, 8, 128, 8, 128), jnp.float32),
    mesh=_mesh,
    compiler_params=pltpu.CompilerParams(
        use_tc_tiling_on_sc=True, needs_layout_passes=False
    ),
    scratch_types=[
        pltpu.VMEM((HIST, 4 * CW), jnp.int32),
        pltpu.VMEM((NBUF, CW), jnp.int32),
        pltpu.VMEM((NBUF, CW, 128), jnp.float32),
        pltpu.VMEM((NBUF, 8, 8, CW), jnp.float32),
        pltpu.SemaphoreType.DMA((NBUF,)),
        pltpu.SemaphoreType.DMA((NBUF,)),
    ],
)
def _gather_kernel(ids_hbm, table_hbm, out_hbm, ibuf, gbuf, rbuf, obuf,
                   gsem, osem):
    wid = lax.axis_index("s") * NC + lax.axis_index("c")
    b0 = wid * (4 * CW)

    iota = jax.lax.iota(jnp.int32, 16)

    for h in range(HIST):
        pltpu.sync_copy(
            ids_hbm.at[pl.ds(h * BATCH + b0, 4 * CW)], ibuf.at[h]
        )

    def shift_ids(s, q):
        h = q // TBW
        j = lax.rem(q, TBW)
        for k in range(CW // 16):
            gbuf[s, pl.ds(16 * k, 16)] = ibuf[h, pl.ds(128 * j + 16 * k, 16)]

    def gather(s):
        return pltpu.make_async_copy(
            table_hbm.at[gbuf.at[s]], rbuf.at[s], gsem.at[s]
        )

    def out_copy(s, q):
        h = q // TBW
        tb = TBW * wid + lax.rem(q, TBW)
        return pltpu.make_async_copy(
            obuf.at[s], out_hbm.at[h, :, tb], osem.at[s]
        )

    def extract(s):
        for m in range(CW // 16):
            rowv = iota + 16 * m

            @plsc.parallel_loop(0, D, unroll=2)
            def _feat(c0):
                cvec = (c0 + iota) & 63
                v = plsc.load_gather(rbuf.at[s], [rowv, cvec])
                plsc.store_scatter(
                    obuf.at[s], [cvec >> 3, cvec & 7, rowv], v
                )

    for s in range(NBUF):
        shift_ids(s, s)
        gather(s).start()

    @pl.loop(0, NCHUNK, step=NBUF)
    def _round(q0):
        for s in range(NBUF):
            q = q0 + s
            gather(s).wait()

            @pl.when(q >= NBUF)
            def _wait_out():
                out_copy(s, q - NBUF).wait()

            extract(s)
            out_copy(s, q).start()
            nq = q + NBUF

            @pl.when(nq < NCHUNK)
            def _refill():
                shift_ids(s, nq)
                gather(s).start()

    for s in range(NBUF):
        out_copy(s, NCHUNK - NBUF + s).wait()


def kernel(token_ids, weight):
    ids_flat = token_ids.T.reshape(B)
    t128 = jnp.pad(weight, ((0, 0), (0, D)))
    out5d = _gather_kernel(ids_flat, t128)
    out = out5d.transpose(2, 4, 0, 1, 3).reshape(BATCH, HIST, D)
    return out
